# baseline (speedup 1.0000x reference)
import functools

import jax
import jax.numpy as jnp
from jax import lax
from jax.experimental import pallas as pl
from jax.experimental.pallas import tpu as pltpu
from jax.experimental.pallas import tpu_sc as plsc

NC = 2
NS = 16
NW = NC * NS

N = 10000
NP = 10240
E = 320000
D = 128
EC = E // NW
K = 80
NCHUNK = EC // K
IDXCH = 2000
ROWS_PER_TILE = NP // NS
BLK = 512
GRID = NP // BLK


def _wid():
  return lax.axis_index("s") * NC + lax.axis_index("c")


def _sc_degrees(src_hbm, dst_hbm, hs_out, hd_out, i0_v, i1_v, hs_v, hd_v,
                isem0, isem1):
  wid = _wid()
  zero16 = jnp.zeros((16,), jnp.float32)
  ones16 = jnp.ones((16,), jnp.float32)
  base = wid * EC
  njobs = EC // IDXCH

  def start_i(j, buf, sem):
    arr = [src_hbm, dst_hbm][j // njobs]
    pltpu.async_copy(arr.at[pl.ds(base + (j % njobs) * IDXCH, IDXCH)], buf,
                     sem)

  def wait_i(buf, sem):
    pltpu.make_async_copy(src_hbm.at[pl.ds(0, IDXCH)], buf, sem).wait()

  start_i(0, i0_v, isem0)

  def zloop(i, _):
    hs_v[pl.ds(i * 16, 16)] = zero16
    hd_v[pl.ds(i * 16, 16)] = zero16
    return 0
  lax.fori_loop(0, NP // 16, zloop, 0)

  bufs = [(i0_v, isem0), (i1_v, isem1)]
  for j in range(2 * njobs):
    buf, sem = bufs[j % 2]
    wait_i(buf, sem)
    if j + 1 < 2 * njobs:
      start_i(j + 1, *bufs[(j + 1) % 2])
    hist = [hs_v, hd_v][j // njobs]

    def inner(k, _):
      idx = buf[pl.ds(k * 16, 16)]
      plsc.addupdate_scatter(hist, [idx], ones16)
      return 0
    lax.fori_loop(0, IDXCH // 16, inner, 0)

  pltpu.sync_copy(hs_v, hs_out.at[wid])
  pltpu.sync_copy(hd_v, hd_out.at[wid])


def _sc_message(h_hbm, src_hbm, dst_hbm, out_hbm, sall_v, d0_v, d1_v, d2_v,
                rows0_v, rows1_v, rows2_v, agg_sh, gsem0, gsem1, gsem2,
                dsem0, dsem1, dsem2):
  cid = lax.axis_index("c")
  sid = lax.axis_index("s")
  wid = sid * NC + cid
  base = wid * EC
  zero16 = jnp.zeros((16,), jnp.float32)

  pltpu.sync_copy(src_hbm.at[pl.ds(base, EC)], sall_v)

  def start_d(c, buf, sem):
    pltpu.async_copy(dst_hbm.at[pl.ds(base + c * K, K)], buf, sem)

  def wait_d(buf, sem):
    pltpu.make_async_copy(dst_hbm.at[pl.ds(0, K)], buf, sem).wait()

  def start_g(c, buf, sem):
    pltpu.async_copy(h_hbm.at[sall_v.at[pl.ds(c * K, K)]], buf, sem)

  def wait_g(buf, sem):
    pltpu.make_async_copy(h_hbm.at[sall_v.at[pl.ds(0, K)]], buf, sem).wait()

  start_d(0, d0_v, dsem0)
  start_d(1, d1_v, dsem1)
  start_d(2, d2_v, dsem2)

  def zrow(i, _):
    def zlane(j, _):
      rows0_v[i, pl.ds(j * 16, 16)] = zero16
      return 0
    lax.fori_loop(0, D // 16, zlane, 0)
    return 0
  lax.fori_loop(0, K, zrow, 0)

  def zagg(i, _):
    pltpu.sync_copy(rows0_v, agg_sh.at[pl.ds(sid * ROWS_PER_TILE + i * K, K)])
    return 0
  lax.fori_loop(0, ROWS_PER_TILE // K, zagg, 0)

  plsc.subcore_barrier()

  start_g(0, rows0_v, gsem0)
  start_g(1, rows1_v, gsem1)

  def step(c, rw, gsw, rn, gsn, dbuf, dsem):
    wait_g(rw, gsw)
    start_g(c + 2, rn, gsn)
    wait_d(dbuf, dsem)
    pltpu.sync_copy(rw, agg_sh.at[dbuf], add=True)

    @pl.when(c + 3 < NCHUNK)
    def _():
      start_d(c + 3, dbuf, dsem)

  def body(i, _):
    c0 = 3 * i
    step(c0, rows0_v, gsem0, rows2_v, gsem2, d0_v, dsem0)
    step(c0 + 1, rows1_v, gsem1, rows0_v, gsem0, d1_v, dsem1)
    step(c0 + 2, rows2_v, gsem2, rows1_v, gsem1, d2_v, dsem2)
    return 0
  lax.fori_loop(0, (NCHUNK - 2) // 3, body, 0)

  wait_g(rows0_v, gsem0)
  wait_d(d0_v, dsem0)
  pltpu.sync_copy(rows0_v, agg_sh.at[d0_v], add=True)
  wait_g(rows1_v, gsem1)
  wait_d(d1_v, dsem1)
  pltpu.sync_copy(rows1_v, agg_sh.at[d1_v], add=True)

  plsc.subcore_barrier()

  pltpu.sync_copy(agg_sh.at[pl.ds(sid * ROWS_PER_TILE, ROWS_PER_TILE)],
                  out_hbm.at[cid, pl.ds(sid * ROWS_PER_TILE, ROWS_PER_TILE)])


_SC_PARAMS = pltpu.CompilerParams(needs_layout_passes=False)

_sc_degrees_call = pl.kernel(
    _sc_degrees,
    out_type=[
        jax.ShapeDtypeStruct((NW, NP), jnp.float32),
        jax.ShapeDtypeStruct((NW, NP), jnp.float32),
    ],
    mesh=plsc.VectorSubcoreMesh(core_axis_name="c", subcore_axis_name="s"),
    compiler_params=_SC_PARAMS,
    scratch_types=[
        pltpu.VMEM((IDXCH,), jnp.int32),
        pltpu.VMEM((IDXCH,), jnp.int32),
        pltpu.VMEM((NP,), jnp.float32),
        pltpu.VMEM((NP,), jnp.float32),
        pltpu.SemaphoreType.DMA,
        pltpu.SemaphoreType.DMA,
    ],
)

_sc_message_call = pl.kernel(
    _sc_message,
    out_type=jax.ShapeDtypeStruct((NC, NP, D), jnp.float32),
    mesh=plsc.VectorSubcoreMesh(core_axis_name="c", subcore_axis_name="s"),
    compiler_params=_SC_PARAMS,
    scratch_types=[
        pltpu.VMEM((EC,), jnp.int32),
        pltpu.VMEM((K,), jnp.int32),
        pltpu.VMEM((K,), jnp.int32),
        pltpu.VMEM((K,), jnp.int32),
        pltpu.VMEM((K, D), jnp.float32),
        pltpu.VMEM((K, D), jnp.float32),
        pltpu.VMEM((K, D), jnp.float32),
        pltpu.VMEM_SHARED((NP, D), jnp.float32),
        pltpu.SemaphoreType.DMA,
        pltpu.SemaphoreType.DMA,
        pltpu.SemaphoreType.DMA,
        pltpu.SemaphoreType.DMA,
        pltpu.SemaphoreType.DMA,
        pltpu.SemaphoreType.DMA,
    ],
)


def _norm_from_hist(hist_blk):
  deg = jnp.sum(hist_blk, axis=0)
  return lax.rsqrt(jnp.clip(deg, 1.0, None))


def _tcmm_body(x_ref, w1_ref, o_ref):
  o_ref[...] = jnp.dot(x_ref[...], w1_ref[...],
                       preferred_element_type=jnp.float32)


def _tcscale_body(h_ref, hs_ref, o_ref):
  norm_src = _norm_from_hist(hs_ref[...])
  o_ref[...] = h_ref[...] * norm_src[:, None]


def _tc2_body(p_ref, hd_ref, hs_ref, b1_ref, w2_ref, o_ref):
  agg = p_ref[0] + p_ref[1]
  norm_dst = _norm_from_hist(hd_ref[...])
  norm_src = _norm_from_hist(hs_ref[...])
  t = jnp.maximum(agg * norm_dst[:, None] + b1_ref[...], 0.0)
  h = jnp.dot(t, w2_ref[...], preferred_element_type=jnp.float32)
  o_ref[...] = h * norm_src[:, None]


def _tc3_body(p_ref, hd_ref, b2_ref, o_ref):
  agg = p_ref[0] + p_ref[1]
  norm_dst = _norm_from_hist(hd_ref[...])
  o_ref[...] = agg * norm_dst[:, None] + b2_ref[...]


_hist_spec = pl.BlockSpec((NW, BLK), lambda i: (0, i))
_row_spec = pl.BlockSpec((BLK, D), lambda i: (i, 0))
_p_spec = pl.BlockSpec((NC, BLK, D), lambda i: (0, i, 0))
_w_spec = pl.BlockSpec((D, D), lambda i: (0, 0))
_b_spec = pl.BlockSpec((1, D), lambda i: (0, 0))

_tcmm_call = pl.pallas_call(
    _tcmm_body,
    grid=(GRID,),
    in_specs=[_row_spec, _w_spec],
    out_specs=_row_spec,
    out_shape=jax.ShapeDtypeStruct((N, D), jnp.float32),
)

_tcscale_call = pl.pallas_call(
    _tcscale_body,
    grid=(GRID,),
    in_specs=[_row_spec, _hist_spec],
    out_specs=_row_spec,
    out_shape=jax.ShapeDtypeStruct((N, D), jnp.float32),
)

_tc2_call = pl.pallas_call(
    _tc2_body,
    grid=(GRID,),
    in_specs=[_p_spec, _hist_spec, _hist_spec, _b_spec, _w_spec],
    out_specs=_row_spec,
    out_shape=jax.ShapeDtypeStruct((N, D), jnp.float32),
)

_tc3_call = pl.pallas_call(
    _tc3_body,
    grid=(GRID,),
    in_specs=[_p_spec, _hist_spec, _b_spec],
    out_specs=_row_spec,
    out_shape=jax.ShapeDtypeStruct((N, D), jnp.float32),
)


@jax.jit
def _run(x, edge_index, W1, b1, W2, b2):
  src = edge_index[0]
  dst = edge_index[1]
  b1r = b1.reshape(1, D)
  b2r = b2.reshape(1, D)

  hm = _tcmm_call(x, W1)
  hs, hd = _sc_degrees_call(src, dst)
  h1 = _tcscale_call(hm, hs)
  p1 = _sc_message_call(h1, src, dst)
  h2 = _tc2_call(p1, hd, hs, b1r, W2)
  p2 = _sc_message_call(h2, src, dst)
  return _tc3_call(p2, hd, b2r)


def kernel(x, edge_index, W1, b1, W2, b2):
  return _run(x, edge_index, W1, b1, W2, b2)


# speedup vs baseline: 1.0048x; 1.0048x over previous
import functools

import jax
import jax.numpy as jnp
from jax import lax
from jax.experimental import pallas as pl
from jax.experimental.pallas import tpu as pltpu
from jax.experimental.pallas import tpu_sc as plsc

NC = 2
NS = 16
NW = NC * NS

N = 10000
NP = 10240
E = 320000
D = 128
EC = E // NW
K = 80
NCHUNK = EC // K
IDXCH = 2000
ROWS_PER_TILE = NP // NS
BLK = 512
GRID = NP // BLK


def _wid():
  return lax.axis_index("s") * NC + lax.axis_index("c")


def _sc_degrees(src_hbm, dst_hbm, hs_out, hd_out, i0_v, i1_v, hs_v, hd_v,
                isem0, isem1):
  wid = _wid()
  zero16 = jnp.zeros((16,), jnp.float32)
  ones16 = jnp.ones((16,), jnp.float32)
  base = wid * EC
  njobs = EC // IDXCH

  def start_i(j, buf, sem):
    arr = [src_hbm, dst_hbm][j // njobs]
    pltpu.async_copy(arr.at[pl.ds(base + (j % njobs) * IDXCH, IDXCH)], buf,
                     sem)

  def wait_i(buf, sem):
    pltpu.make_async_copy(src_hbm.at[pl.ds(0, IDXCH)], buf, sem).wait()

  start_i(0, i0_v, isem0)

  def zloop(i, _):
    hs_v[pl.ds(i * 16, 16)] = zero16
    hd_v[pl.ds(i * 16, 16)] = zero16
    return 0
  lax.fori_loop(0, NP // 16, zloop, 0)

  bufs = [(i0_v, isem0), (i1_v, isem1)]
  for j in range(2 * njobs):
    buf, sem = bufs[j % 2]
    wait_i(buf, sem)
    if j + 1 < 2 * njobs:
      start_i(j + 1, *bufs[(j + 1) % 2])
    hist = [hs_v, hd_v][j // njobs]

    def inner(k, _):
      idx = buf[pl.ds(k * 16, 16)]
      plsc.addupdate_scatter(hist, [idx], ones16)
      return 0
    lax.fori_loop(0, IDXCH // 16, inner, 0)

  pltpu.sync_copy(hs_v, hs_out.at[wid])
  pltpu.sync_copy(hd_v, hd_out.at[wid])


def _sc_message(h_hbm, src_hbm, dst_hbm, out_hbm, sall_v, d0_v, d1_v, d2_v,
                rows0_v, rows1_v, rows2_v, agg_sh, gsem0, gsem1, gsem2,
                dsem0, dsem1, dsem2):
  cid = lax.axis_index("c")
  sid = lax.axis_index("s")
  wid = sid * NC + cid
  base = wid * EC
  zero16 = jnp.zeros((16,), jnp.float32)

  pltpu.sync_copy(src_hbm.at[pl.ds(base, EC)], sall_v)

  def start_d(c, buf, sem):
    pltpu.async_copy(dst_hbm.at[pl.ds(base + c * K, K)], buf, sem)

  def wait_d(buf, sem):
    pltpu.make_async_copy(dst_hbm.at[pl.ds(0, K)], buf, sem).wait()

  def start_g(c, buf, sem):
    pltpu.async_copy(h_hbm.at[sall_v.at[pl.ds(c * K, K)]], buf, sem)

  def wait_g(buf, sem):
    pltpu.make_async_copy(h_hbm.at[sall_v.at[pl.ds(0, K)]], buf, sem).wait()

  start_d(0, d0_v, dsem0)
  start_d(1, d1_v, dsem1)
  start_d(2, d2_v, dsem2)

  def zrow(i, _):
    def zlane(j, _):
      rows0_v[i, pl.ds(j * 16, 16)] = zero16
      return 0
    lax.fori_loop(0, D // 16, zlane, 0)
    return 0
  lax.fori_loop(0, K, zrow, 0)

  def zagg(i, _):
    pltpu.sync_copy(rows0_v, agg_sh.at[pl.ds(sid * ROWS_PER_TILE + i * K, K)])
    return 0
  lax.fori_loop(0, ROWS_PER_TILE // K, zagg, 0)

  plsc.subcore_barrier()

  start_g(0, rows0_v, gsem0)
  start_g(1, rows1_v, gsem1)

  def step(c, rw, gsw, rn, gsn, dbuf, dsem):
    wait_g(rw, gsw)
    start_g(c + 2, rn, gsn)
    wait_d(dbuf, dsem)
    pltpu.sync_copy(rw, agg_sh.at[dbuf], add=True)

    @pl.when(c + 3 < NCHUNK)
    def _():
      start_d(c + 3, dbuf, dsem)

  def body(i, _):
    c0 = 3 * i
    step(c0, rows0_v, gsem0, rows2_v, gsem2, d0_v, dsem0)
    step(c0 + 1, rows1_v, gsem1, rows0_v, gsem0, d1_v, dsem1)
    step(c0 + 2, rows2_v, gsem2, rows1_v, gsem1, d2_v, dsem2)
    return 0
  lax.fori_loop(0, (NCHUNK - 2) // 3, body, 0)

  wait_g(rows0_v, gsem0)
  wait_d(d0_v, dsem0)
  pltpu.sync_copy(rows0_v, agg_sh.at[d0_v], add=True)
  wait_g(rows1_v, gsem1)
  wait_d(d1_v, dsem1)
  pltpu.sync_copy(rows1_v, agg_sh.at[d1_v], add=True)

  plsc.subcore_barrier()

  pltpu.sync_copy(agg_sh.at[pl.ds(sid * ROWS_PER_TILE, ROWS_PER_TILE)],
                  out_hbm.at[cid, pl.ds(sid * ROWS_PER_TILE, ROWS_PER_TILE)])


_SC_PARAMS = pltpu.CompilerParams(needs_layout_passes=False)

_sc_degrees_call = pl.kernel(
    _sc_degrees,
    out_type=[
        jax.ShapeDtypeStruct((NW, NP), jnp.float32),
        jax.ShapeDtypeStruct((NW, NP), jnp.float32),
    ],
    mesh=plsc.VectorSubcoreMesh(core_axis_name="c", subcore_axis_name="s"),
    compiler_params=_SC_PARAMS,
    scratch_types=[
        pltpu.VMEM((IDXCH,), jnp.int32),
        pltpu.VMEM((IDXCH,), jnp.int32),
        pltpu.VMEM((NP,), jnp.float32),
        pltpu.VMEM((NP,), jnp.float32),
        pltpu.SemaphoreType.DMA,
        pltpu.SemaphoreType.DMA,
    ],
)

_sc_message_call = pl.kernel(
    _sc_message,
    out_type=jax.ShapeDtypeStruct((NC, NP, D), jnp.float32),
    mesh=plsc.VectorSubcoreMesh(core_axis_name="c", subcore_axis_name="s"),
    compiler_params=_SC_PARAMS,
    scratch_types=[
        pltpu.VMEM((EC,), jnp.int32),
        pltpu.VMEM((K,), jnp.int32),
        pltpu.VMEM((K,), jnp.int32),
        pltpu.VMEM((K,), jnp.int32),
        pltpu.VMEM((K, D), jnp.float32),
        pltpu.VMEM((K, D), jnp.float32),
        pltpu.VMEM((K, D), jnp.float32),
        pltpu.VMEM_SHARED((NP, D), jnp.float32),
        pltpu.SemaphoreType.DMA,
        pltpu.SemaphoreType.DMA,
        pltpu.SemaphoreType.DMA,
        pltpu.SemaphoreType.DMA,
        pltpu.SemaphoreType.DMA,
        pltpu.SemaphoreType.DMA,
    ],
)


def _norm_from_hist(hist_blk):
  deg = jnp.sum(hist_blk, axis=0)
  return lax.rsqrt(jnp.clip(deg, 1.0, None))


def _tc1_body(x_ref, hs_ref, w1_ref, o_ref):
  norm_src = _norm_from_hist(hs_ref[...])
  h = jnp.dot(x_ref[...], w1_ref[...], preferred_element_type=jnp.float32)
  o_ref[...] = h * norm_src[:, None]


def _tc2_body(p_ref, hd_ref, hs_ref, b1_ref, w2_ref, o_ref):
  agg = p_ref[0] + p_ref[1]
  norm_dst = _norm_from_hist(hd_ref[...])
  norm_src = _norm_from_hist(hs_ref[...])
  t = jnp.maximum(agg * norm_dst[:, None] + b1_ref[...], 0.0)
  h = jnp.dot(t, w2_ref[...], preferred_element_type=jnp.float32)
  o_ref[...] = h * norm_src[:, None]


def _tc3_body(p_ref, hd_ref, b2_ref, o_ref):
  agg = p_ref[0] + p_ref[1]
  norm_dst = _norm_from_hist(hd_ref[...])
  o_ref[...] = agg * norm_dst[:, None] + b2_ref[...]


_hist_spec = pl.BlockSpec((NW, BLK), lambda i: (0, i))
_row_spec = pl.BlockSpec((BLK, D), lambda i: (i, 0))
_p_spec = pl.BlockSpec((NC, BLK, D), lambda i: (0, i, 0))
_w_spec = pl.BlockSpec((D, D), lambda i: (0, 0))
_b_spec = pl.BlockSpec((1, D), lambda i: (0, 0))

_tc1_call = pl.pallas_call(
    _tc1_body,
    grid=(GRID,),
    in_specs=[_row_spec, _hist_spec, _w_spec],
    out_specs=_row_spec,
    out_shape=jax.ShapeDtypeStruct((N, D), jnp.float32),
)

_tc2_call = pl.pallas_call(
    _tc2_body,
    grid=(GRID,),
    in_specs=[_p_spec, _hist_spec, _hist_spec, _b_spec, _w_spec],
    out_specs=_row_spec,
    out_shape=jax.ShapeDtypeStruct((N, D), jnp.float32),
)

_tc3_call = pl.pallas_call(
    _tc3_body,
    grid=(GRID,),
    in_specs=[_p_spec, _hist_spec, _b_spec],
    out_specs=_row_spec,
    out_shape=jax.ShapeDtypeStruct((N, D), jnp.float32),
)


@jax.jit
def _run(x, edge_index, W1, b1, W2, b2):
  src = edge_index[0]
  dst = edge_index[1]
  b1r = b1.reshape(1, D)
  b2r = b2.reshape(1, D)

  hs, hd = _sc_degrees_call(src, dst)
  h1 = _tc1_call(x, hs, W1)
  p1 = _sc_message_call(h1, src, dst)
  h2 = _tc2_call(p1, hd, hs, b1r, W2)
  p2 = _sc_message_call(h2, src, dst)
  return _tc3_call(p2, hd, b2r)


def kernel(x, edge_index, W1, b1, W2, b2):
  return _run(x, edge_index, W1, b1, W2, b2)
